# baseline (speedup 1.0000x reference)
import functools

import jax
import jax.numpy as jnp
from jax import lax
from jax.experimental import pallas as pl
from jax.experimental.pallas import tpu as pltpu
from jax.experimental.pallas import tpu_sc as plsc

_F = 26
_V = 100000
_D = 32
_B = 4096

_NC = 2
_NS = 16
_NW = _NC * _NS
_ROWS = _B * _F
_RPW = _ROWS // _NW
_IR = _RPW // 128


def _sc_gather(cat2, tab):
    mesh = plsc.VectorSubcoreMesh(core_axis_name="c", subcore_axis_name="s")

    @functools.partial(
        pl.kernel,
        mesh=mesh,
        out_type=jax.ShapeDtypeStruct((_ROWS, _D), jnp.float32),
        scratch_types=[
            pltpu.VMEM((_RPW,), jnp.int32),
            pltpu.VMEM((_RPW, _D), jnp.float32),
            pltpu.SemaphoreType.DMA,
        ],
        compiler_params=pltpu.CompilerParams(use_tc_tiling_on_sc=False),
    )
    def body(cat_hbm, tab_hbm, out_hbm, idx_v, rows_v, sem):
        wid = lax.axis_index("s") * _NC + lax.axis_index("c")
        pltpu.sync_copy(cat_hbm.at[pl.ds(wid * _RPW, _RPW)], idx_v)

        iota = lax.iota(jnp.int32, 16)
        offv = [((j * 16 + iota) % _F) * _V for j in range(13)]
        for q in range(_RPW // 16):
            sl = pl.ds(q * 16, 16)
            idx_v[sl] = idx_v[sl] + offv[q % 13]

        copies = []
        for r in range(_IR):
            copies.append(
                pltpu.async_copy(
                    tab_hbm.at[idx_v.at[pl.ds(r * 128, 128)]],
                    rows_v.at[pl.ds(r * 128, 128)],
                    sem,
                )
            )
        for c in copies:
            c.wait()

        pltpu.sync_copy(rows_v, out_hbm.at[pl.ds(wid * _RPW, _RPW)])

    return body(cat2, tab)


def kernel(categorical_data, tables):
    cat2 = categorical_data.astype(jnp.int32).reshape(_ROWS)
    tab = tables.reshape(_F * _V, _D)
    out = _sc_gather(cat2, tab)
    return out.reshape(_B, _F * _D)


# speedup vs baseline: 5.3282x; 5.3282x over previous
import functools

import jax
import jax.numpy as jnp
from jax import lax
from jax.experimental import pallas as pl
from jax.experimental.pallas import tpu as pltpu
from jax.experimental.pallas import tpu_sc as plsc

_F = 26
_V = 100000
_D = 32
_B = 4096

_NC = 2
_NS = 16
_NW = _NC * _NS
_R = _F * _D
_RPW = _R // _NW


def _sc_gather(cat_t, tab2):
    mesh = plsc.VectorSubcoreMesh(core_axis_name="c", subcore_axis_name="s")

    @functools.partial(
        pl.kernel,
        mesh=mesh,
        out_type=jax.ShapeDtypeStruct((_R, _B), jnp.float32),
        scratch_types=[
            pltpu.VMEM((_V,), jnp.float32),
            pltpu.VMEM((_B,), jnp.int32),
            pltpu.VMEM((_B,), jnp.float32),
        ],
        compiler_params=pltpu.CompilerParams(
            use_tc_tiling_on_sc=True, needs_layout_passes=False
        ),
    )
    def body(cat_hbm, tab_hbm, out_hbm, row_v, idx_v, out_v, *, _=None):
        wid = lax.axis_index("s") * _NC + lax.axis_index("c")
        base = wid * _RPW

        def do_row(rl, carry):
            r = base + rl
            f = lax.shift_right_logical(r, 5)
            pltpu.sync_copy(cat_hbm.at[f], idx_v)
            pltpu.sync_copy(tab_hbm.at[r], row_v)

            def gather16(j, c):
                sl = pl.ds(pl.multiple_of(j * 16, 16), 16)
                idx_v16 = idx_v[sl]
                out_v[sl] = plsc.load_gather(row_v, [idx_v16])
                return c

            lax.fori_loop(0, _B // 16, gather16, 0, unroll=4)
            pltpu.sync_copy(out_v, out_hbm.at[r])
            return carry

        lax.fori_loop(0, _RPW, do_row, 0)

    return body(cat_t, tab2)


def kernel(categorical_data, tables):
    cat_t = categorical_data.astype(jnp.int32).T
    tab2 = tables.transpose(0, 2, 1).reshape(_R, _V)
    out2 = _sc_gather(cat_t, tab2)
    return out2.T.reshape(_B, _F * _D)


# speedup vs baseline: 5.5436x; 1.0404x over previous
import functools

import jax
import jax.numpy as jnp
from jax import lax
from jax.experimental import pallas as pl
from jax.experimental.pallas import tpu as pltpu
from jax.experimental.pallas import tpu_sc as plsc

_F = 26
_V = 100000
_D = 32
_B = 4096

_NC = 2
_NS = 16
_NW = _NC * _NS
_R = _F * _D
_RPW = _R // _NW
_VA = 50048
_VB = _V - _VA


def _sc_gather(cat_t, tab2):
    mesh = plsc.VectorSubcoreMesh(core_axis_name="c", subcore_axis_name="s")

    @functools.partial(
        pl.kernel,
        mesh=mesh,
        out_type=jax.ShapeDtypeStruct((_R, _B), jnp.float32),
        scratch_types=[
            pltpu.VMEM((_VA,), jnp.float32),
            pltpu.VMEM((_VB,), jnp.float32),
            pltpu.VMEM((_B,), jnp.int32),
            pltpu.VMEM((_B,), jnp.float32),
            pltpu.SemaphoreType.DMA,
            pltpu.SemaphoreType.DMA,
            pltpu.SemaphoreType.DMA,
            pltpu.SemaphoreType.DMA,
        ],
        compiler_params=pltpu.CompilerParams(
            use_tc_tiling_on_sc=True, needs_layout_passes=False
        ),
    )
    def body(cat_hbm, tab_hbm, out_hbm, buf_a, buf_b, idx_v, out_v,
             sem_i, sem_a, sem_b, sem_o):
        wid = lax.axis_index("s") * _NC + lax.axis_index("c")
        base = wid * _RPW
        dsa = pl.ds(0, _VA)
        dsb = pl.ds(_VA, _VB)

        f0 = lax.shift_right_logical(base, 5)
        pltpu.async_copy(cat_hbm.at[f0], idx_v, sem_i)
        pltpu.async_copy(tab_hbm.at[base, dsa], buf_a, sem_a)
        pltpu.async_copy(tab_hbm.at[base, dsb], buf_b, sem_b)

        def gather_half(buf, first):
            lo = 0 if first else _VA

            def one(j, c):
                sl = pl.ds(pl.multiple_of(j * 16, 16), 16)
                v = idx_v[sl]
                m = (v < _VA) if first else (v >= _VA)
                vv = jnp.where(m, v - lo, 0)
                g = plsc.load_gather(buf, [vv])
                contrib = jnp.where(m, g, jnp.float32(0))
                out_v[sl] = contrib if first else out_v[sl] + contrib
                return c

            lax.fori_loop(0, _B // 16, one, 0, unroll=4)

        def do_row(rl, carry):
            r = base + rl
            not_last = rl < _RPW - 1
            rn = jnp.where(not_last, r + 1, r)

            @pl.when(rl > 0)
            def _():
                pltpu.make_async_copy(out_v, out_hbm.at[r], sem_o).wait()

            f = lax.shift_right_logical(r, 5)
            pltpu.make_async_copy(cat_hbm.at[f], idx_v, sem_i).wait()
            pltpu.make_async_copy(tab_hbm.at[r, dsa], buf_a, sem_a).wait()
            gather_half(buf_a, True)

            @pl.when(not_last)
            def _():
                pltpu.async_copy(tab_hbm.at[rn, dsa], buf_a, sem_a)

            pltpu.make_async_copy(tab_hbm.at[r, dsb], buf_b, sem_b).wait()
            gather_half(buf_b, False)

            @pl.when(not_last)
            def _():
                fn = lax.shift_right_logical(rn, 5)
                pltpu.async_copy(cat_hbm.at[fn], idx_v, sem_i)

            pltpu.async_copy(out_v, out_hbm.at[r], sem_o)

            @pl.when(not_last)
            def _():
                pltpu.async_copy(tab_hbm.at[rn, dsb], buf_b, sem_b)

            return carry

        lax.fori_loop(0, _RPW, do_row, 0)
        pltpu.make_async_copy(out_v, out_hbm.at[base], sem_o).wait()

    return body(cat_t, tab2)


def kernel(categorical_data, tables):
    cat_t = categorical_data.astype(jnp.int32).T
    tab2 = tables.transpose(0, 2, 1).reshape(_R, _V)
    out2 = _sc_gather(cat_t, tab2)
    return out2.T.reshape(_B, _F * _D)


# speedup vs baseline: 6.5931x; 1.1893x over previous
import functools

import jax
import jax.numpy as jnp
from jax import lax
from jax.experimental import pallas as pl
from jax.experimental.pallas import tpu as pltpu
from jax.experimental.pallas import tpu_sc as plsc

_F = 26
_V = 100000
_D = 32
_B = 4096

_NC = 2
_NS = 16
_NW = _NC * _NS
_R = _F * _D
_RPW = _R // _NW
_VA = 50048
_VB = _V - _VA


def _sc_gather(cat_t, tab2):
    mesh = plsc.VectorSubcoreMesh(core_axis_name="c", subcore_axis_name="s")

    @functools.partial(
        pl.kernel,
        mesh=mesh,
        out_type=jax.ShapeDtypeStruct((_R, _B), jnp.float32),
        scratch_types=[
            pltpu.VMEM((_VA,), jnp.float32),
            pltpu.VMEM((_VB,), jnp.float32),
            pltpu.VMEM((_B,), jnp.int32),
            pltpu.VMEM((_B,), jnp.float32),
            pltpu.SemaphoreType.DMA,
            pltpu.SemaphoreType.DMA,
            pltpu.SemaphoreType.DMA,
            pltpu.SemaphoreType.DMA,
        ],
        compiler_params=pltpu.CompilerParams(
            use_tc_tiling_on_sc=True, needs_layout_passes=False
        ),
    )
    def body(cat_hbm, tab_hbm, out_hbm, buf_a, buf_b, idx_v, out_v,
             sem_i, sem_a, sem_b, sem_o):
        wid = lax.axis_index("s") * _NC + lax.axis_index("c")
        base = wid * _RPW
        dsa = pl.ds(0, _VA)
        dsb = pl.ds(_VA, _VB)

        f0 = lax.shift_right_logical(base, 5)
        pltpu.async_copy(cat_hbm.at[f0], idx_v, sem_i)
        pltpu.async_copy(tab_hbm.at[base, dsa], buf_a, sem_a)
        pltpu.async_copy(tab_hbm.at[base, dsb], buf_b, sem_b)

        def gather_half(buf, first):
            lo = 0 if first else _VA

            def one(j, c):
                sl = pl.ds(pl.multiple_of(j * 16, 16), 16)
                v = idx_v[sl]
                m = (v < _VA) if first else (v >= _VA)
                g = plsc.load_gather(buf, [v - lo], mask=m)
                contrib = jnp.where(m, g, jnp.float32(0))
                out_v[sl] = contrib if first else out_v[sl] + contrib
                return c

            lax.fori_loop(0, _B // 16, one, 0, unroll=4)

        def do_row(rl, carry):
            r = base + rl
            not_last = rl < _RPW - 1
            rn = jnp.where(not_last, r + 1, r)
            f = lax.shift_right_logical(r, 5)
            fn = lax.shift_right_logical(rn, 5)

            @pl.when(rl > 0)
            def _():
                pltpu.make_async_copy(out_v, out_hbm.at[r], sem_o).wait()

            @pl.when((rl == 0) | (f != lax.shift_right_logical(r - 1, 5)))
            def _():
                pltpu.make_async_copy(cat_hbm.at[f], idx_v, sem_i).wait()

            pltpu.make_async_copy(tab_hbm.at[r, dsa], buf_a, sem_a).wait()
            gather_half(buf_a, True)

            @pl.when(not_last)
            def _():
                pltpu.async_copy(tab_hbm.at[rn, dsa], buf_a, sem_a)

            pltpu.make_async_copy(tab_hbm.at[r, dsb], buf_b, sem_b).wait()
            gather_half(buf_b, False)

            @pl.when(not_last & (fn != f))
            def _():
                pltpu.async_copy(cat_hbm.at[fn], idx_v, sem_i)

            pltpu.async_copy(out_v, out_hbm.at[r], sem_o)

            @pl.when(not_last)
            def _():
                pltpu.async_copy(tab_hbm.at[rn, dsb], buf_b, sem_b)

            return carry

        lax.fori_loop(0, _RPW, do_row, 0)
        pltpu.make_async_copy(out_v, out_hbm.at[base], sem_o).wait()

    return body(cat_t, tab2)


def kernel(categorical_data, tables):
    cat_t = categorical_data.astype(jnp.int32).T
    tab2 = tables.transpose(0, 2, 1).reshape(_R, _V)
    out2 = _sc_gather(cat_t, tab2)
    return out2.T.reshape(_B, _F * _D)
